# baseline (speedup 1.0000x reference)
import functools

import jax
import jax.numpy as jnp
from jax import lax
from jax.experimental import pallas as pl
from jax.experimental.pallas import tpu as pltpu
from jax.experimental.pallas import tpu_sc as plsc

F, B, L = 26, 4096, 20
VOCAB, DIM = 100000, 32
LANES = 16
NC, NS = 2, 16
NW = NC * NS

BAGS = F * B
BAGS_PER_W = BAGS // NW
CHUNK_BAGS = 64
N_CHUNKS = BAGS_PER_W // CHUNK_BAGS
N_PAIRS = N_CHUNKS // 2
CHUNKS_PER_F = B // CHUNK_BAGS
VPB = CHUNK_BAGS // LANES


def _pool_body(idx_hbm, tbl_hbm, out_hbm,
               raw_v, idx_v0, idx_v1, rows_v0, rows_v1, out_v,
               sem0, sem1):
    wid = lax.axis_index("s") * NC + lax.axis_index("c")
    lane_iota = lax.iota(jnp.int32, LANES)

    def coords(k):
        g = wid * N_CHUNKS + k
        f = g // CHUNKS_PER_F
        b0 = (g % CHUNKS_PER_F) * CHUNK_BAGS
        return f, b0

    def launch(k, idx_v, rows_v, sem):
        f, b0 = coords(k)
        pltpu.sync_copy(idx_hbm.at[f, :, pl.ds(b0, CHUNK_BAGS)], raw_v)
        off_vec = jnp.full((LANES,), f * VOCAB, dtype=jnp.int32)

        def remap_body(t, _):
            r = t // (2 * VPB)
            c = (t % (2 * VPB)) * LANES
            rr = t // VPB
            cc = (t % VPB) * LANES
            idx_v[r, pl.ds(c, LANES)] = lax.rem(raw_v[rr, pl.ds(cc, LANES)],
                                                VOCAB) + off_vec
            return 0

        lax.fori_loop(0, L * VPB, remap_body, 0)
        for j in range(L // 2):
            pltpu.async_copy(tbl_hbm.at[idx_v.at[j]], rows_v.at[j], sem)

    def finish(k, idx_v, rows_v, sem):
        f, b0 = coords(k)
        for j in range(L // 2):
            pltpu.make_async_copy(tbl_hbm.at[idx_v.at[j]], rows_v.at[j],
                                  sem).wait()

        def bag_body(b, _):
            a0 = rows_v[0, b, pl.ds(0, LANES)]
            a1 = rows_v[0, b, pl.ds(LANES, LANES)]
            for l in range(1, L):
                r, c = l // 2, (l % 2) * CHUNK_BAGS
                a0 = a0 + rows_v[r, c + b, pl.ds(0, LANES)]
                a1 = a1 + rows_v[r, c + b, pl.ds(LANES, LANES)]
            b_vec = jnp.full((LANES,), b, dtype=jnp.int32)
            plsc.store_scatter(out_v, [lane_iota, b_vec], a0)
            plsc.store_scatter(out_v, [lane_iota + LANES, b_vec], a1)
            return 0

        lax.fori_loop(0, CHUNK_BAGS, bag_body, 0)
        pltpu.sync_copy(out_v.at[:, pl.ds(0, CHUNK_BAGS)],
                        out_hbm.at[f, :, pl.ds(b0, CHUNK_BAGS)])

    launch(0, idx_v0, rows_v0, sem0)

    def pair_body(p, _):
        c0 = 2 * p
        launch(c0 + 1, idx_v1, rows_v1, sem1)
        finish(c0, idx_v0, rows_v0, sem0)

        @pl.when(c0 + 2 < N_CHUNKS)
        def _():
            launch(c0 + 2, idx_v0, rows_v0, sem0)

        finish(c0 + 1, idx_v1, rows_v1, sem1)
        return 0

    lax.fori_loop(0, N_PAIRS, pair_body, 0)


@jax.jit
def kernel(indices, tables):
    idx_t = indices.transpose(0, 2, 1)
    tbl_flat = tables.reshape(F * VOCAB, DIM)

    mesh = plsc.VectorSubcoreMesh(core_axis_name="c", subcore_axis_name="s",
                                  num_cores=NC, num_subcores=NS)
    run = functools.partial(
        pl.kernel,
        out_type=jax.ShapeDtypeStruct((F, DIM, B), jnp.float32),
        mesh=mesh,
        scratch_types=[
            pltpu.VMEM((L, CHUNK_BAGS), jnp.int32),
            pltpu.VMEM((L // 2, 2 * CHUNK_BAGS), jnp.int32),
            pltpu.VMEM((L // 2, 2 * CHUNK_BAGS), jnp.int32),
            pltpu.VMEM((L // 2, 2 * CHUNK_BAGS, DIM), jnp.float32),
            pltpu.VMEM((L // 2, 2 * CHUNK_BAGS, DIM), jnp.float32),
            pltpu.VMEM((DIM, CHUNK_BAGS + 1), jnp.float32),
            pltpu.SemaphoreType.DMA,
            pltpu.SemaphoreType.DMA,
        ],
        compiler_params=pltpu.CompilerParams(use_tc_tiling_on_sc=False,
                                             needs_layout_passes=False),
    )(_pool_body)
    out_t = run(idx_t, tbl_flat)
    return out_t.transpose(0, 2, 1)


# speedup vs baseline: 1.0556x; 1.0556x over previous
import functools

import jax
import jax.numpy as jnp
from jax import lax
from jax.experimental import pallas as pl
from jax.experimental.pallas import tpu as pltpu
from jax.experimental.pallas import tpu_sc as plsc

F, B, L = 26, 4096, 20
VOCAB, DIM = 100000, 32
LANES = 16
NC, NS = 2, 16
NW = NC * NS

BAGS = F * B
BAGS_PER_W = BAGS // NW
CHUNK_BAGS = 64
N_CHUNKS = BAGS_PER_W // CHUNK_BAGS
N_PAIRS = N_CHUNKS // 2
CHUNKS_PER_F = B // CHUNK_BAGS
VPB = CHUNK_BAGS // LANES


def _pool_body(idx_hbm, tbl_hbm, out_hbm,
               raw_v, idx_v0, idx_v1, rows_v0, rows_v1, out_v,
               sem0, sem1):
    wid = lax.axis_index("s") * NC + lax.axis_index("c")
    lane_iota = lax.iota(jnp.int32, LANES)

    def coords(k):
        g = wid * N_CHUNKS + k
        f = g // CHUNKS_PER_F
        b0 = (g % CHUNKS_PER_F) * CHUNK_BAGS
        return f, b0

    def launch(k, idx_v, rows_v, sem):
        f, b0 = coords(k)
        pltpu.sync_copy(idx_hbm.at[f, :, pl.ds(b0, CHUNK_BAGS)], raw_v)
        off_vec = jnp.full((LANES,), f * VOCAB, dtype=jnp.int32)

        def remap_body(t, _):
            r = t // VPB
            c = (t % VPB) * LANES
            idx_v[r, pl.ds(c, LANES)] = lax.rem(raw_v[r, pl.ds(c, LANES)],
                                                VOCAB) + off_vec
            return 0

        lax.fori_loop(0, L * VPB, remap_body, 0)
        for l in range(L):
            pltpu.async_copy(tbl_hbm.at[idx_v.at[l]], rows_v.at[l], sem)

    def finish(k, idx_v, rows_v, sem):
        f, b0 = coords(k)
        for l in range(L):
            pltpu.make_async_copy(tbl_hbm.at[idx_v.at[l]], rows_v.at[l],
                                  sem).wait()

        def bag_body(b, _):
            a0 = rows_v[0, b, pl.ds(0, LANES)]
            a1 = rows_v[0, b, pl.ds(LANES, LANES)]
            for l in range(1, L):
                a0 = a0 + rows_v[l, b, pl.ds(0, LANES)]
                a1 = a1 + rows_v[l, b, pl.ds(LANES, LANES)]
            b_vec = jnp.full((LANES,), b, dtype=jnp.int32)
            plsc.store_scatter(out_v, [lane_iota, b_vec], a0)
            plsc.store_scatter(out_v, [lane_iota + LANES, b_vec], a1)
            return 0

        lax.fori_loop(0, CHUNK_BAGS, bag_body, 0)
        pltpu.sync_copy(out_v.at[:, pl.ds(0, CHUNK_BAGS)],
                        out_hbm.at[f, :, pl.ds(b0, CHUNK_BAGS)])

    launch(0, idx_v0, rows_v0, sem0)

    def pair_body(p, _):
        c0 = 2 * p
        launch(c0 + 1, idx_v1, rows_v1, sem1)
        finish(c0, idx_v0, rows_v0, sem0)

        @pl.when(c0 + 2 < N_CHUNKS)
        def _():
            launch(c0 + 2, idx_v0, rows_v0, sem0)

        finish(c0 + 1, idx_v1, rows_v1, sem1)
        return 0

    lax.fori_loop(0, N_PAIRS, pair_body, 0)


@jax.jit
def kernel(indices, tables):
    idx_t = indices.transpose(0, 2, 1)
    tbl_flat = tables.reshape(F * VOCAB, DIM)

    mesh = plsc.VectorSubcoreMesh(core_axis_name="c", subcore_axis_name="s",
                                  num_cores=NC, num_subcores=NS)
    run = functools.partial(
        pl.kernel,
        out_type=jax.ShapeDtypeStruct((F, DIM, B), jnp.float32),
        mesh=mesh,
        scratch_types=[
            pltpu.VMEM((L, CHUNK_BAGS), jnp.int32),
            pltpu.VMEM((L, CHUNK_BAGS), jnp.int32),
            pltpu.VMEM((L, CHUNK_BAGS), jnp.int32),
            pltpu.VMEM((L, CHUNK_BAGS, DIM), jnp.float32),
            pltpu.VMEM((L, CHUNK_BAGS, DIM), jnp.float32),
            pltpu.VMEM((DIM, CHUNK_BAGS + 1), jnp.float32),
            pltpu.SemaphoreType.DMA,
            pltpu.SemaphoreType.DMA,
        ],
        compiler_params=pltpu.CompilerParams(use_tc_tiling_on_sc=False,
                                             needs_layout_passes=False),
    )(_pool_body)
    out_t = run(idx_t, tbl_flat)
    return out_t.transpose(0, 2, 1)
